# Initial kernel scaffold; baseline (speedup 1.0000x reference)
import functools

import jax
import jax.numpy as jnp
from jax import lax
from jax.experimental import pallas as pl
from jax.experimental.pallas import tpu as pltpu
from jax.experimental.pallas import tpu_sc as plsc

N = 10000
E = 320000
F = 128
NH = 8
HD = 16
L = 16
NC, NS = 2, 16
NW = NC * NS
EPW = E // NW
CH = 80
NIT = EPW // CH
RPW = N // NS
_OFFS = (0, 80, 160, 240, 320, 400, 480, 545)
_EPS = 1e-9

_MESH = plsc.VectorSubcoreMesh(core_axis_name="c", subcore_axis_name="s",
                               num_cores=NC, num_subcores=NS)

_GDN = lax.GatherDimensionNumbers(offset_dims=(), collapsed_slice_dims=(0,),
                                  start_index_map=(0,))


def _bcast_lane(v, lane):
    idx = jnp.full((L, 1), lane, dtype=jnp.int32)
    return lax.gather(v, idx, _GDN, (1,),
                      mode=lax.GatherScatterMode.PROMISE_IN_BOUNDS)


def _expleaky(e):
    return jnp.exp(jnp.where(e >= 0.0, e, 0.2 * e))


def _edge_body(heads, h_hbm, elr_hbm, src_hbm, dst_hbm, msg_out, s_out,
               msg_sh, s_sh, src_v, dst_v, el_v, er_v, rows_v, srow_v, sem):
    cid = lax.axis_index("c")
    sid = lax.axis_index("s")
    wid = sid * NC + cid
    base = wid * EPW
    iota = lax.iota(jnp.int32, L)
    zf = jnp.zeros((L,), jnp.float32)

    def _zr(i, _):
        for h in range(F // L):
            rows_v[i, pl.ds(h * L, L)] = zf
        srow_v[i, pl.ds(0, L)] = zf
        return 0
    lax.fori_loop(0, CH, _zr, 0)
    r0 = sid * RPW
    for o in _OFFS:
        pltpu.sync_copy(rows_v, msg_sh.at[pl.ds(r0 + o, CH)])
        pltpu.sync_copy(srow_v, s_sh.at[pl.ds(r0 + o, CH)])
    plsc.subcore_barrier()

    def _iter(j, _):
        eb = base + j * CH
        pltpu.sync_copy(src_hbm.at[pl.ds(eb, CH)], src_v)
        pltpu.sync_copy(dst_hbm.at[pl.ds(eb, CH)], dst_v)
        cps = [pltpu.async_copy(h_hbm.at[src_v], rows_v, sem),
               pltpu.async_copy(elr_hbm.at[src_v], el_v, sem),
               pltpu.async_copy(elr_hbm.at[dst_v], er_v, sem)]
        for cp in cps:
            cp.wait()
        for g in range(CH // L):
            j0 = g * L
            if heads == NH:
                wv = []
                for r in range(NH):
                    le = j0 + 2 * r + iota // NH
                    lh = iota % NH
                    ev = (plsc.load_gather(el_v, [le, lh]) +
                          plsc.load_gather(er_v, [le, lh + NH]))
                    w = _expleaky(ev)
                    wv.append(w)
                    plsc.store_scatter(srow_v, [le, lh], w)
                for e in range(L):
                    for h in range(NH):
                        p = NH * e + h
                        wb = _bcast_lane(wv[p // L], p % L)
                        r = rows_v[j0 + e, pl.ds(h * HD, HD)]
                        rows_v[j0 + e, pl.ds(h * HD, HD)] = r * wb
            else:
                el2 = plsc.load_gather(el_v, [j0 + iota, iota * 0])
                er2 = plsc.load_gather(er_v, [j0 + iota, iota * 0 + 1])
                w2 = _expleaky(el2 + er2)
                plsc.store_scatter(srow_v, [j0 + iota, iota * 0], w2)
                for e in range(L):
                    wb = _bcast_lane(w2, e)
                    for h in range(F // L):
                        r = rows_v[j0 + e, pl.ds(h * L, L)]
                        rows_v[j0 + e, pl.ds(h * L, L)] = r * wb
        pltpu.sync_copy(rows_v, msg_sh.at[dst_v], add=True)
        pltpu.sync_copy(srow_v, s_sh.at[dst_v], add=True)
        return 0
    lax.fori_loop(0, NIT, _iter, 0)

    plsc.subcore_barrier()
    for o in _OFFS:
        pltpu.sync_copy(msg_sh.at[pl.ds(r0 + o, CH)], rows_v)
        pltpu.sync_copy(rows_v, msg_out.at[pl.ds(cid * N + r0 + o, CH)])
        pltpu.sync_copy(s_sh.at[pl.ds(r0 + o, CH)], srow_v)
        pltpu.sync_copy(srow_v, s_out.at[pl.ds(cid * N + r0 + o, CH)])


def _make_edge_kernel(heads):
    return pl.kernel(
        functools.partial(_edge_body, heads),
        out_type=(jax.ShapeDtypeStruct((NC * N, F), jnp.float32),
                  jax.ShapeDtypeStruct((NC * N, L), jnp.float32)),
        mesh=_MESH,
        scratch_types=[
            pltpu.VMEM_SHARED((N, F), jnp.float32),
            pltpu.VMEM_SHARED((N, L), jnp.float32),
            pltpu.VMEM((CH,), jnp.int32),
            pltpu.VMEM((CH,), jnp.int32),
            pltpu.VMEM((CH, L), jnp.float32),
            pltpu.VMEM((CH, L), jnp.float32),
            pltpu.VMEM((CH, F), jnp.float32),
            pltpu.VMEM((CH, L), jnp.float32),
            pltpu.SemaphoreType.DMA,
        ],
        name=f"gat_edge_h{heads}",
    )


_edge1 = _make_edge_kernel(NH)
_edge2 = _make_edge_kernel(1)


def _score_body(hf_hbm, src_hbm, dst_hbm, out_hbm,
                src_v, dst_v, a_v, b_v, dot_v, out_v, sem):
    cid = lax.axis_index("c")
    sid = lax.axis_index("s")
    base = (sid * NC + cid) * EPW
    iota = lax.iota(jnp.int32, L)

    def _iter(j, _):
        eb = base + j * CH
        pltpu.sync_copy(src_hbm.at[pl.ds(eb, CH)], src_v)
        pltpu.sync_copy(dst_hbm.at[pl.ds(eb, CH)], dst_v)
        c1 = pltpu.async_copy(hf_hbm.at[src_v], a_v, sem)
        c2 = pltpu.async_copy(hf_hbm.at[dst_v], b_v, sem)
        c1.wait()
        c2.wait()
        for g in range(CH // L):
            j0 = g * L
            for e in range(L):
                acc = a_v[j0 + e, pl.ds(0, L)] * b_v[j0 + e, pl.ds(0, L)]
                for h in range(1, F // L):
                    acc = acc + (a_v[j0 + e, pl.ds(h * L, L)] *
                                 b_v[j0 + e, pl.ds(h * L, L)])
                dot_v[e, pl.ds(0, L)] = acc
            tot = plsc.load_gather(dot_v, [iota, iota * 0])
            for k in range(1, L):
                tot = tot + plsc.load_gather(dot_v, [iota, iota * 0 + k])
            out_v[pl.ds(j0, L)] = 1.0 / (1.0 + jnp.exp(-tot))
        pltpu.sync_copy(out_v, out_hbm.at[pl.ds(eb, CH)])
        return 0
    lax.fori_loop(0, NIT, _iter, 0)


_score = pl.kernel(
    _score_body,
    out_type=jax.ShapeDtypeStruct((E,), jnp.float32),
    mesh=_MESH,
    scratch_types=[
        pltpu.VMEM((CH,), jnp.int32),
        pltpu.VMEM((CH,), jnp.int32),
        pltpu.VMEM((CH, F), jnp.float32),
        pltpu.VMEM((CH, F), jnp.float32),
        pltpu.VMEM((L, L), jnp.float32),
        pltpu.VMEM((CH,), jnp.float32),
        pltpu.SemaphoreType.DMA,
    ],
    name="gat_score",
)


def _dense1_body(x_ref, w_ref, a_ref, h_ref, elr_ref):
    h = jnp.dot(x_ref[...], w_ref[...], preferred_element_type=jnp.float32)
    h_ref[...] = h
    elr_ref[...] = jnp.dot(h, a_ref[...], preferred_element_type=jnp.float32)


_dense1 = pl.pallas_call(
    _dense1_body,
    out_shape=(jax.ShapeDtypeStruct((N, F), jnp.float32),
               jax.ShapeDtypeStruct((N, L), jnp.float32)),
)


def _dense2_body(msg_ref, s_ref, r_ref, b_ref, w_ref, a_ref, h_ref, elr_ref):
    msg = msg_ref[pl.ds(0, N), :] + msg_ref[pl.ds(N, N), :]
    s = s_ref[pl.ds(0, N), :] + s_ref[pl.ds(N, N), :]
    srep = jnp.dot(s, r_ref[...], preferred_element_type=jnp.float32)
    x2 = jnp.maximum(msg / (srep + _EPS) + b_ref[...], 0.0)
    h2 = jnp.dot(x2, w_ref[...], preferred_element_type=jnp.float32)
    h_ref[...] = h2
    elr_ref[...] = jnp.dot(h2, a_ref[...], preferred_element_type=jnp.float32)


_dense2 = pl.pallas_call(
    _dense2_body,
    out_shape=(jax.ShapeDtypeStruct((N, F), jnp.float32),
               jax.ShapeDtypeStruct((N, L), jnp.float32)),
)


def _dense3_body(msg_ref, s_ref, r_ref, b_ref, hf_ref):
    msg = msg_ref[pl.ds(0, N), :] + msg_ref[pl.ds(N, N), :]
    s = s_ref[pl.ds(0, N), :] + s_ref[pl.ds(N, N), :]
    srep = jnp.dot(s, r_ref[...], preferred_element_type=jnp.float32)
    hf_ref[...] = jnp.maximum(msg / (srep + _EPS) + b_ref[...], 0.0)


_dense3 = pl.pallas_call(
    _dense3_body,
    out_shape=jax.ShapeDtypeStruct((N, F), jnp.float32),
)


def kernel(features, edge_index, edge_type, W1, a_l1, a_r1, b1,
           W2, a_l2, a_r2, b2):
    del edge_type
    src = edge_index[0]
    dst = edge_index[1]
    eye8 = jnp.eye(NH, dtype=jnp.float32)
    Al = (a_l1[:, :, None] * eye8[:, None, :]).reshape(F, NH)
    Ar = (a_r1[:, :, None] * eye8[:, None, :]).reshape(F, NH)
    A1 = jnp.concatenate([Al, Ar], axis=1)
    A2 = (jnp.zeros((F, L), jnp.float32)
          .at[:, 0].set(a_l2[0]).at[:, 1].set(a_r2[0]))
    R1 = jnp.concatenate(
        [jnp.kron(eye8, jnp.ones((1, HD), jnp.float32)),
         jnp.zeros((NH, F), jnp.float32)], axis=0)
    R2 = jnp.zeros((L, F), jnp.float32).at[0].set(1.0)

    h1, elr1 = _dense1(features, W1, A1)
    msg1, s1 = _edge1(h1, elr1, src, dst)
    h2, elr2 = _dense2(msg1, s1, R1, b1.reshape(1, F), W2, A2)
    msg2, s2 = _edge2(h2, elr2, src, dst)
    hf = _dense3(msg2, s2, R2, b2.reshape(1, F))
    return _score(hf, src, dst)


# speedup vs baseline: 11.7725x; 11.7725x over previous
import jax
import jax.numpy as jnp
from jax import lax
from jax.experimental import pallas as pl
from jax.experimental.pallas import tpu as pltpu
from jax.experimental.pallas import tpu_sc as plsc

N = 10000
E = 320000
F = 128
NH = 8
HD = 16
L = 16
NC, NS = 2, 16
NW = NC * NS
EPW = E // NW
SEPW = E // NS
CH = 80
CH2 = 40
NIT = EPW // CH
NIT2 = EPW // CH2
NITW = SEPW // CH
NP = 10240
RPW = NP // NS
_OFFS = tuple(range(0, RPW, CH))
_OFFS2 = tuple(range(0, RPW, CH2))
HH = NH // NC
_EPS = 1e-9

_MESH = plsc.VectorSubcoreMesh(core_axis_name="c", subcore_axis_name="s",
                               num_cores=NC, num_subcores=NS)
_SC_PARAMS = pltpu.CompilerParams(needs_layout_passes=False)


def _expleaky(e):
    return jnp.exp(jnp.where(e >= 0.0, e, 0.2 * e))


def _w1_body(elh_hbm, erh_hbm, src_hbm, dst_hbm, w_out, s_out,
             el_v, er_v, s4_v, src_v, dst_v, w_v, sem):
    cid = lax.axis_index("c")
    sid = lax.axis_index("s")
    iota = lax.iota(jnp.int32, L)
    zf = (iota * 0).astype(jnp.float32)
    tb = cid * (HH * N)
    pltpu.sync_copy(elh_hbm.at[pl.ds(tb, HH * N)], el_v)
    pltpu.sync_copy(erh_hbm.at[pl.ds(tb, HH * N)], er_v)

    def _zs(i, _):
        s4_v[pl.ds(i * L, L)] = zf
        return 0
    lax.fori_loop(0, HH * N // L, _zs, 0)
    base = sid * SEPW

    def _iter(j, _):
        eb = base + j * CH
        pltpu.sync_copy(src_hbm.at[pl.ds(eb, CH)], src_v)
        pltpu.sync_copy(dst_hbm.at[pl.ds(eb, CH)], dst_v)
        for q in range(CH * HH // L):
            p0 = q * L
            le = p0 // HH + iota // HH
            lh = iota % HH
            sg = plsc.load_gather(src_v, [le])
            dg = plsc.load_gather(dst_v, [le])
            elv = plsc.load_gather(el_v, [sg * HH + lh])
            erv = plsc.load_gather(er_v, [dg * HH + lh])
            w = _expleaky(elv + erv)
            w_v[pl.ds(p0, L)] = w
            didx = dg * HH + lh
            for e4 in range(HH):
                plsc.addupdate_scatter(s4_v, [didx], w,
                                       mask=iota // HH == e4)
        pltpu.sync_copy(w_v, w_out.at[pl.ds(cid * (HH * E) + eb * HH,
                                            HH * CH)])
        return 0
    lax.fori_loop(0, NITW, _iter, 0)
    wid = cid * NS + sid
    pltpu.sync_copy(s4_v, s_out.at[pl.ds(wid * (HH * N), HH * N)])


_wpass1 = pl.kernel(
    _w1_body,
    out_type=(jax.ShapeDtypeStruct((NC * HH * E,), jnp.float32),
              jax.ShapeDtypeStruct((NW * HH * N,), jnp.float32)),
    mesh=_MESH,
    compiler_params=_SC_PARAMS,
    scratch_types=[
        pltpu.VMEM((HH * N,), jnp.float32),
        pltpu.VMEM((HH * N,), jnp.float32),
        pltpu.VMEM((HH * N,), jnp.float32),
        pltpu.VMEM((CH,), jnp.int32),
        pltpu.VMEM((CH,), jnp.int32),
        pltpu.VMEM((HH * CH,), jnp.float32),
        pltpu.SemaphoreType.DMA,
    ],
    name="gat_w1",
)


CHE = 512
NITE = E // CHE


def _main1_body(h1t_hbm, w_hbm, src_hbm, dst_hbm, msgt_out,
                ht_v, acc_v, src_v, dst_v, w_v, sem):
    cid = lax.axis_index("c")
    sid = lax.axis_index("s")
    wid = sid * NC + cid
    head = wid // 4
    hw = head % HH
    hf_sel = head // HH
    iota = lax.iota(jnp.int32, L)
    zf = (iota * 0).astype(jnp.float32)
    pltpu.sync_copy(h1t_hbm.at[pl.ds(wid * (4 * N), 4 * N)], ht_v)

    def _za(i, _):
        acc_v[pl.ds(i * L, L)] = zf
        return 0
    lax.fori_loop(0, 4 * N // L, _za, 0)

    def _iter(j, _):
        eb = j * CHE
        pltpu.sync_copy(src_hbm.at[pl.ds(eb, CHE)], src_v)
        pltpu.sync_copy(dst_hbm.at[pl.ds(eb, CHE)], dst_v)
        pltpu.sync_copy(
            w_hbm.at[pl.ds(hf_sel * (HH * E) + HH * eb, HH * CHE)], w_v)
        for g in range(CHE // L):
            j0 = g * L
            sg = plsc.load_gather(src_v, [j0 + iota])
            dg = plsc.load_gather(dst_v, [j0 + iota])
            wv = plsc.load_gather(w_v, [HH * (j0 + iota) + hw])
            for cl in range(4):
                hv = plsc.load_gather(ht_v, [cl * N + sg])
                plsc.addupdate_scatter(acc_v, [cl * N + dg], hv * wv)
        return 0
    lax.fori_loop(0, NITE, _iter, 0)
    pltpu.sync_copy(acc_v, msgt_out.at[pl.ds(wid * (4 * N), 4 * N)])


_main1 = pl.kernel(
    _main1_body,
    out_type=jax.ShapeDtypeStruct((F * N,), jnp.float32),
    mesh=_MESH,
    compiler_params=_SC_PARAMS,
    scratch_types=[
        pltpu.VMEM((4 * N,), jnp.float32),
        pltpu.VMEM((4 * N,), jnp.float32),
        pltpu.VMEM((CHE,), jnp.int32),
        pltpu.VMEM((CHE,), jnp.int32),
        pltpu.VMEM((HH * CHE,), jnp.float32),
        pltpu.SemaphoreType.DMA,
    ],
    name="gat_main1",
)


def _main2_body(h2t_hbm, el_hbm, er_hbm, src_hbm, dst_hbm, msgt_out, s_out,
                ht_v, acc_v, el_v, er_v, s2_v, src_v, dst_v, sem):
    cid = lax.axis_index("c")
    sid = lax.axis_index("s")
    wid = sid * NC + cid
    iota = lax.iota(jnp.int32, L)
    zf = (iota * 0).astype(jnp.float32)
    pltpu.sync_copy(h2t_hbm.at[pl.ds(wid * (4 * N), 4 * N)], ht_v)
    pltpu.sync_copy(el_hbm, el_v)
    pltpu.sync_copy(er_hbm, er_v)

    def _za(i, _):
        acc_v[pl.ds(i * L, L)] = zf
        return 0
    lax.fori_loop(0, 4 * N // L, _za, 0)

    def _zs(i, _):
        s2_v[pl.ds(i * L, L)] = zf
        return 0
    lax.fori_loop(0, N // L, _zs, 0)

    def _iter(j, _):
        eb = j * CHE
        pltpu.sync_copy(src_hbm.at[pl.ds(eb, CHE)], src_v)
        pltpu.sync_copy(dst_hbm.at[pl.ds(eb, CHE)], dst_v)
        for g in range(CHE // L):
            j0 = g * L
            sg = plsc.load_gather(src_v, [j0 + iota])
            dg = plsc.load_gather(dst_v, [j0 + iota])
            w2 = _expleaky(plsc.load_gather(el_v, [sg]) +
                           plsc.load_gather(er_v, [dg]))
            plsc.addupdate_scatter(s2_v, [dg], w2)
            for cl in range(4):
                hv = plsc.load_gather(ht_v, [cl * N + sg])
                plsc.addupdate_scatter(acc_v, [cl * N + dg], hv * w2)
        return 0
    lax.fori_loop(0, NITE, _iter, 0)
    pltpu.sync_copy(acc_v, msgt_out.at[pl.ds(wid * (4 * N), 4 * N)])
    pltpu.sync_copy(s2_v, s_out.at[pl.ds(wid * N, N)])


_main2 = pl.kernel(
    _main2_body,
    out_type=(jax.ShapeDtypeStruct((F * N,), jnp.float32),
              jax.ShapeDtypeStruct((NW * N,), jnp.float32)),
    mesh=_MESH,
    compiler_params=_SC_PARAMS,
    scratch_types=[
        pltpu.VMEM((4 * N,), jnp.float32),
        pltpu.VMEM((4 * N,), jnp.float32),
        pltpu.VMEM((N,), jnp.float32),
        pltpu.VMEM((N,), jnp.float32),
        pltpu.VMEM((N,), jnp.float32),
        pltpu.VMEM((CHE,), jnp.int32),
        pltpu.VMEM((CHE,), jnp.int32),
        pltpu.SemaphoreType.DMA,
    ],
    name="gat_main2",
)


def _score_body(hf_hbm, src_hbm, dst_hbm, out_hbm,
                src_v, dst_v, a_v, b_v, dot_v, out_v, sem):
    cid = lax.axis_index("c")
    sid = lax.axis_index("s")
    base = (sid * NC + cid) * EPW
    iota = lax.iota(jnp.int32, L)

    def _iter(j, _):
        eb = base + j * CH
        pltpu.sync_copy(src_hbm.at[pl.ds(eb, CH)], src_v)
        pltpu.sync_copy(dst_hbm.at[pl.ds(eb, CH)], dst_v)
        c1 = pltpu.async_copy(hf_hbm.at[src_v], a_v, sem)
        c2 = pltpu.async_copy(hf_hbm.at[dst_v], b_v, sem)
        c1.wait()
        c2.wait()
        for g in range(CH // L):
            j0 = g * L
            for e in range(L):
                acc = a_v[j0 + e, pl.ds(0, L)] * b_v[j0 + e, pl.ds(0, L)]
                for h in range(1, F // L):
                    acc = acc + (a_v[j0 + e, pl.ds(h * L, L)] *
                                 b_v[j0 + e, pl.ds(h * L, L)])
                dot_v[e, pl.ds(0, L)] = acc
            tot = plsc.load_gather(dot_v, [iota, iota * 0])
            for k in range(1, L):
                tot = tot + plsc.load_gather(dot_v, [iota, iota * 0 + k])
            out_v[pl.ds(j0, L)] = 1.0 / (1.0 + jnp.exp(-tot))
        pltpu.sync_copy(out_v, out_hbm.at[pl.ds(eb, CH)])
        return 0
    lax.fori_loop(0, NIT, _iter, 0)


_score = pl.kernel(
    _score_body,
    out_type=jax.ShapeDtypeStruct((E,), jnp.float32),
    mesh=_MESH,
    compiler_params=_SC_PARAMS,
    scratch_types=[
        pltpu.VMEM((CH,), jnp.int32),
        pltpu.VMEM((CH,), jnp.int32),
        pltpu.VMEM((CH, F), jnp.float32),
        pltpu.VMEM((CH, F), jnp.float32),
        pltpu.VMEM((L, L), jnp.float32),
        pltpu.VMEM((CH,), jnp.float32),
        pltpu.SemaphoreType.DMA,
    ],
    name="gat_score",
)


def _dense1_body(x_ref, w_ref, a_ref, h_ref, elr_ref):
    h = jnp.dot(x_ref[...], w_ref[...], preferred_element_type=jnp.float32)
    h_ref[...] = h
    elr_ref[...] = jnp.dot(h, a_ref[...], preferred_element_type=jnp.float32)


_dense1 = pl.pallas_call(
    _dense1_body,
    out_shape=(jax.ShapeDtypeStruct((N, F), jnp.float32),
               jax.ShapeDtypeStruct((N, L), jnp.float32)),
)


def _red1_body(a_ref, b_ref, oa_ref, ob_ref):
    oa_ref[...] = jnp.sum(a_ref[...], axis=0)
    ob_ref[...] = jnp.sum(b_ref[...], axis=0)


_red1 = pl.pallas_call(
    _red1_body,
    out_shape=(jax.ShapeDtypeStruct((HH * N,), jnp.float32),
               jax.ShapeDtypeStruct((HH * N,), jnp.float32)),
)


def _dense2_body(msg_ref, s_ref, r_ref, b_ref, w_ref, a_ref, h_ref, elr_ref):
    msg = msg_ref[...]
    srep = jnp.dot(s_ref[...], r_ref[...], preferred_element_type=jnp.float32)
    x2 = jnp.maximum(msg / (srep + _EPS) + b_ref[...], 0.0)
    h2 = jnp.dot(x2, w_ref[...], preferred_element_type=jnp.float32)
    h_ref[...] = h2
    elr_ref[...] = jnp.dot(h2, a_ref[...], preferred_element_type=jnp.float32)


_dense2 = pl.pallas_call(
    _dense2_body,
    out_shape=(jax.ShapeDtypeStruct((N, F), jnp.float32),
               jax.ShapeDtypeStruct((N, L), jnp.float32)),
)


def _dense3_body(msg_ref, s_ref, r_ref, b_ref, hf_ref):
    msg = msg_ref[...]
    srep = jnp.dot(s_ref[...], r_ref[...], preferred_element_type=jnp.float32)
    hf_ref[...] = jnp.maximum(msg / (srep + _EPS) + b_ref[...], 0.0)


_dense3 = pl.pallas_call(
    _dense3_body,
    out_shape=jax.ShapeDtypeStruct((N, F), jnp.float32),
)


def kernel(features, edge_index, edge_type, W1, a_l1, a_r1, b1,
           W2, a_l2, a_r2, b2):
    del edge_type
    src = edge_index[0]
    dst = edge_index[1]
    eye8 = jnp.eye(NH, dtype=jnp.float32)
    Al = (a_l1[:, :, None] * eye8[:, None, :]).reshape(F, NH)
    Ar = (a_r1[:, :, None] * eye8[:, None, :]).reshape(F, NH)
    A1 = jnp.concatenate([Al, Ar], axis=1)
    A2 = (jnp.zeros((F, L), jnp.float32)
          .at[:, 0].set(a_l2[0]).at[:, 1].set(a_r2[0]))
    R1 = jnp.kron(eye8, jnp.ones((1, HD), jnp.float32))
    R2 = jnp.ones((1, F), jnp.float32)

    h1, elr1 = _dense1(features, W1, A1)
    el1 = elr1[:, :NH]
    er1 = elr1[:, NH:]
    elh = jnp.concatenate([el1[:, :HH].reshape(-1), el1[:, HH:].reshape(-1)])
    erh = jnp.concatenate([er1[:, :HH].reshape(-1), er1[:, HH:].reshape(-1)])
    w1, s4p = _wpass1(elh, erh, src, dst)
    s4m = s4p.reshape(NW, HH * N)
    s4a, s4b = _red1(s4m[:NS], s4m[NS:])
    s1 = jnp.concatenate([s4a.reshape(N, HH), s4b.reshape(N, HH)], axis=1)
    msg1t = _main1(h1.T.reshape(-1), w1, src, dst)
    msg1 = msg1t.reshape(F, N).T
    h2, elr2 = _dense2(msg1, s1, R1, b1.reshape(1, F), W2, A2)
    msg2t, s2p = _main2(h2.T.reshape(-1), elr2[:, 0], elr2[:, 1], src, dst)
    msg2 = msg2t.reshape(F, N).T
    s2 = s2p[:N]
    hf = _dense3(msg2, s2.reshape(N, 1), R2, b2.reshape(1, F))
    return _score(hf, src, dst)


# speedup vs baseline: 13.9094x; 1.1815x over previous
import jax
import jax.numpy as jnp
from jax import lax
from jax.experimental import pallas as pl
from jax.experimental.pallas import tpu as pltpu
from jax.experimental.pallas import tpu_sc as plsc

N = 10000
E = 320000
F = 128
NH = 8
HD = 16
L = 16
NC, NS = 2, 16
NW = NC * NS
EPW = E // NW
SEPW = E // NS
CH = 80
CH2 = 40
NIT = EPW // CH
NIT2 = EPW // CH2
NITW = SEPW // CH
NP = 10240
RPW = NP // NS
_OFFS = tuple(range(0, RPW, CH))
_OFFS2 = tuple(range(0, RPW, CH2))
HH = NH // NC
_EPS = 1e-9

_MESH = plsc.VectorSubcoreMesh(core_axis_name="c", subcore_axis_name="s",
                               num_cores=NC, num_subcores=NS)
_SC_PARAMS = pltpu.CompilerParams(needs_layout_passes=False)


def _expleaky(e):
    return jnp.exp(jnp.where(e >= 0.0, e, 0.2 * e))


def _w1_body(elh_hbm, erh_hbm, src_hbm, dst_hbm, w_out, s_out,
             el_v, er_v, s4_v, src_v, dst_v, w_v, sem):
    cid = lax.axis_index("c")
    sid = lax.axis_index("s")
    iota = lax.iota(jnp.int32, L)
    zf = (iota * 0).astype(jnp.float32)
    tb = cid * (HH * N)
    pltpu.sync_copy(elh_hbm.at[pl.ds(tb, HH * N)], el_v)
    pltpu.sync_copy(erh_hbm.at[pl.ds(tb, HH * N)], er_v)

    def _zs(i, _):
        s4_v[pl.ds(i * L, L)] = zf
        return 0
    lax.fori_loop(0, HH * N // L, _zs, 0)
    base = sid * SEPW

    def _iter(j, _):
        eb = base + j * CH
        pltpu.sync_copy(src_hbm.at[pl.ds(eb, CH)], src_v)
        pltpu.sync_copy(dst_hbm.at[pl.ds(eb, CH)], dst_v)
        for q in range(CH * HH // L):
            p0 = q * L
            le = p0 // HH + iota // HH
            lh = iota % HH
            sg = plsc.load_gather(src_v, [le])
            dg = plsc.load_gather(dst_v, [le])
            elv = plsc.load_gather(el_v, [sg * HH + lh])
            erv = plsc.load_gather(er_v, [dg * HH + lh])
            w = _expleaky(elv + erv)
            w_v[pl.ds(p0, L)] = w
            didx = dg * HH + lh
            for e4 in range(HH):
                plsc.addupdate_scatter(s4_v, [didx], w,
                                       mask=iota // HH == e4)
        pltpu.sync_copy(w_v, w_out.at[pl.ds(cid * (HH * E) + eb * HH,
                                            HH * CH)])
        return 0
    lax.fori_loop(0, NITW, _iter, 0)
    wid = cid * NS + sid
    pltpu.sync_copy(s4_v, s_out.at[pl.ds(wid * (HH * N), HH * N)])


_wpass1 = pl.kernel(
    _w1_body,
    out_type=(jax.ShapeDtypeStruct((NC * HH * E,), jnp.float32),
              jax.ShapeDtypeStruct((NW * HH * N,), jnp.float32)),
    mesh=_MESH,
    compiler_params=_SC_PARAMS,
    scratch_types=[
        pltpu.VMEM((HH * N,), jnp.float32),
        pltpu.VMEM((HH * N,), jnp.float32),
        pltpu.VMEM((HH * N,), jnp.float32),
        pltpu.VMEM((CH,), jnp.int32),
        pltpu.VMEM((CH,), jnp.int32),
        pltpu.VMEM((HH * CH,), jnp.float32),
        pltpu.SemaphoreType.DMA,
    ],
    name="gat_w1",
)


CHE = 3200
NITE = E // CHE


def _main1_body(h1t_hbm, w_hbm, src_hbm, dst_hbm, msgt_out,
                ht_v, acc_v, src_v, dst_v, w_v, sem):
    cid = lax.axis_index("c")
    sid = lax.axis_index("s")
    wid = sid * NC + cid
    head = wid // 4
    hw = head % HH
    hf_sel = head // HH
    iota = lax.iota(jnp.int32, L)
    zf = (iota * 0).astype(jnp.float32)
    pltpu.sync_copy(h1t_hbm.at[pl.ds(wid * (4 * N), 4 * N)], ht_v)

    def _za(i, _):
        acc_v[pl.ds(i * L, L)] = zf
        return 0
    lax.fori_loop(0, 4 * N // L, _za, 0)

    def _iter(j, _):
        eb = j * CHE
        c1 = pltpu.async_copy(src_hbm.at[pl.ds(eb, CHE)], src_v, sem)
        c2 = pltpu.async_copy(dst_hbm.at[pl.ds(eb, CHE)], dst_v, sem)
        c3 = pltpu.async_copy(
            w_hbm.at[pl.ds(hf_sel * (HH * E) + HH * eb, HH * CHE)], w_v, sem)
        c1.wait()
        c2.wait()
        c3.wait()
        for g in range(CHE // L):
            j0 = g * L
            sg = plsc.load_gather(src_v, [j0 + iota])
            dg = plsc.load_gather(dst_v, [j0 + iota])
            wv = plsc.load_gather(w_v, [HH * (j0 + iota) + hw])
            for cl in range(4):
                hv = plsc.load_gather(ht_v, [cl * N + sg])
                plsc.addupdate_scatter(acc_v, [cl * N + dg], hv * wv)
        return 0
    lax.fori_loop(0, NITE, _iter, 0)
    pltpu.sync_copy(acc_v, msgt_out.at[pl.ds(wid * (4 * N), 4 * N)])


_main1 = pl.kernel(
    _main1_body,
    out_type=jax.ShapeDtypeStruct((F * N,), jnp.float32),
    mesh=_MESH,
    compiler_params=_SC_PARAMS,
    scratch_types=[
        pltpu.VMEM((4 * N,), jnp.float32),
        pltpu.VMEM((4 * N,), jnp.float32),
        pltpu.VMEM((CHE,), jnp.int32),
        pltpu.VMEM((CHE,), jnp.int32),
        pltpu.VMEM((HH * CHE,), jnp.float32),
        pltpu.SemaphoreType.DMA,
    ],
    name="gat_main1",
)


def _main2_body(h2t_hbm, el_hbm, er_hbm, src_hbm, dst_hbm, msgt_out, s_out,
                ht_v, acc_v, el_v, er_v, s2_v, src_v, dst_v, sem):
    cid = lax.axis_index("c")
    sid = lax.axis_index("s")
    wid = sid * NC + cid
    iota = lax.iota(jnp.int32, L)
    zf = (iota * 0).astype(jnp.float32)
    pltpu.sync_copy(h2t_hbm.at[pl.ds(wid * (4 * N), 4 * N)], ht_v)
    pltpu.sync_copy(el_hbm, el_v)
    pltpu.sync_copy(er_hbm, er_v)

    def _za(i, _):
        acc_v[pl.ds(i * L, L)] = zf
        return 0
    lax.fori_loop(0, 4 * N // L, _za, 0)

    def _zs(i, _):
        s2_v[pl.ds(i * L, L)] = zf
        return 0
    lax.fori_loop(0, N // L, _zs, 0)

    def _iter(j, _):
        eb = j * CHE
        c1 = pltpu.async_copy(src_hbm.at[pl.ds(eb, CHE)], src_v, sem)
        c2 = pltpu.async_copy(dst_hbm.at[pl.ds(eb, CHE)], dst_v, sem)
        c1.wait()
        c2.wait()
        for g in range(CHE // L):
            j0 = g * L
            sg = plsc.load_gather(src_v, [j0 + iota])
            dg = plsc.load_gather(dst_v, [j0 + iota])
            w2 = _expleaky(plsc.load_gather(el_v, [sg]) +
                           plsc.load_gather(er_v, [dg]))
            plsc.addupdate_scatter(s2_v, [dg], w2)
            for cl in range(4):
                hv = plsc.load_gather(ht_v, [cl * N + sg])
                plsc.addupdate_scatter(acc_v, [cl * N + dg], hv * w2)
        return 0
    lax.fori_loop(0, NITE, _iter, 0)
    pltpu.sync_copy(acc_v, msgt_out.at[pl.ds(wid * (4 * N), 4 * N)])
    pltpu.sync_copy(s2_v, s_out.at[pl.ds(wid * N, N)])


_main2 = pl.kernel(
    _main2_body,
    out_type=(jax.ShapeDtypeStruct((F * N,), jnp.float32),
              jax.ShapeDtypeStruct((NW * N,), jnp.float32)),
    mesh=_MESH,
    compiler_params=_SC_PARAMS,
    scratch_types=[
        pltpu.VMEM((4 * N,), jnp.float32),
        pltpu.VMEM((4 * N,), jnp.float32),
        pltpu.VMEM((N,), jnp.float32),
        pltpu.VMEM((N,), jnp.float32),
        pltpu.VMEM((N,), jnp.float32),
        pltpu.VMEM((CHE,), jnp.int32),
        pltpu.VMEM((CHE,), jnp.int32),
        pltpu.SemaphoreType.DMA,
    ],
    name="gat_main2",
)


def _score_body(hf_hbm, src_hbm, dst_hbm, out_hbm,
                src_v, dst_v, a_v, b_v, dot_v, out_v, sem):
    cid = lax.axis_index("c")
    sid = lax.axis_index("s")
    base = (sid * NC + cid) * EPW
    iota = lax.iota(jnp.int32, L)

    def _iter(j, _):
        eb = base + j * CH
        pltpu.sync_copy(src_hbm.at[pl.ds(eb, CH)], src_v)
        pltpu.sync_copy(dst_hbm.at[pl.ds(eb, CH)], dst_v)
        c1 = pltpu.async_copy(hf_hbm.at[src_v], a_v, sem)
        c2 = pltpu.async_copy(hf_hbm.at[dst_v], b_v, sem)
        c1.wait()
        c2.wait()
        for g in range(CH // L):
            j0 = g * L
            for e in range(L):
                acc = a_v[j0 + e, pl.ds(0, L)] * b_v[j0 + e, pl.ds(0, L)]
                for h in range(1, F // L):
                    acc = acc + (a_v[j0 + e, pl.ds(h * L, L)] *
                                 b_v[j0 + e, pl.ds(h * L, L)])
                dot_v[e, pl.ds(0, L)] = acc
            tot = plsc.load_gather(dot_v, [iota, iota * 0])
            for k in range(1, L):
                tot = tot + plsc.load_gather(dot_v, [iota, iota * 0 + k])
            out_v[pl.ds(j0, L)] = 1.0 / (1.0 + jnp.exp(-tot))
        pltpu.sync_copy(out_v, out_hbm.at[pl.ds(eb, CH)])
        return 0
    lax.fori_loop(0, NIT, _iter, 0)


_score = pl.kernel(
    _score_body,
    out_type=jax.ShapeDtypeStruct((E,), jnp.float32),
    mesh=_MESH,
    compiler_params=_SC_PARAMS,
    scratch_types=[
        pltpu.VMEM((CH,), jnp.int32),
        pltpu.VMEM((CH,), jnp.int32),
        pltpu.VMEM((CH, F), jnp.float32),
        pltpu.VMEM((CH, F), jnp.float32),
        pltpu.VMEM((L, L), jnp.float32),
        pltpu.VMEM((CH,), jnp.float32),
        pltpu.SemaphoreType.DMA,
    ],
    name="gat_score",
)


def _dense1_body(x_ref, w_ref, a_ref, h_ref, elr_ref):
    h = jnp.dot(x_ref[...], w_ref[...], preferred_element_type=jnp.float32)
    h_ref[...] = h
    elr_ref[...] = jnp.dot(h, a_ref[...], preferred_element_type=jnp.float32)


_dense1 = pl.pallas_call(
    _dense1_body,
    out_shape=(jax.ShapeDtypeStruct((N, F), jnp.float32),
               jax.ShapeDtypeStruct((N, L), jnp.float32)),
)


def _red1_body(a_ref, b_ref, oa_ref, ob_ref):
    oa_ref[...] = jnp.sum(a_ref[...], axis=0)
    ob_ref[...] = jnp.sum(b_ref[...], axis=0)


_red1 = pl.pallas_call(
    _red1_body,
    out_shape=(jax.ShapeDtypeStruct((HH * N,), jnp.float32),
               jax.ShapeDtypeStruct((HH * N,), jnp.float32)),
)


def _dense2_body(msg_ref, s_ref, r_ref, b_ref, w_ref, a_ref, h_ref, elr_ref):
    msg = msg_ref[...]
    srep = jnp.dot(s_ref[...], r_ref[...], preferred_element_type=jnp.float32)
    x2 = jnp.maximum(msg / (srep + _EPS) + b_ref[...], 0.0)
    h2 = jnp.dot(x2, w_ref[...], preferred_element_type=jnp.float32)
    h_ref[...] = h2
    elr_ref[...] = jnp.dot(h2, a_ref[...], preferred_element_type=jnp.float32)


_dense2 = pl.pallas_call(
    _dense2_body,
    out_shape=(jax.ShapeDtypeStruct((N, F), jnp.float32),
               jax.ShapeDtypeStruct((N, L), jnp.float32)),
)


def _dense3_body(msg_ref, s_ref, r_ref, b_ref, hf_ref):
    msg = msg_ref[...]
    srep = jnp.dot(s_ref[...], r_ref[...], preferred_element_type=jnp.float32)
    hf_ref[...] = jnp.maximum(msg / (srep + _EPS) + b_ref[...], 0.0)


_dense3 = pl.pallas_call(
    _dense3_body,
    out_shape=jax.ShapeDtypeStruct((N, F), jnp.float32),
)


def kernel(features, edge_index, edge_type, W1, a_l1, a_r1, b1,
           W2, a_l2, a_r2, b2):
    del edge_type
    src = edge_index[0]
    dst = edge_index[1]
    eye8 = jnp.eye(NH, dtype=jnp.float32)
    Al = (a_l1[:, :, None] * eye8[:, None, :]).reshape(F, NH)
    Ar = (a_r1[:, :, None] * eye8[:, None, :]).reshape(F, NH)
    A1 = jnp.concatenate([Al, Ar], axis=1)
    A2 = (jnp.zeros((F, L), jnp.float32)
          .at[:, 0].set(a_l2[0]).at[:, 1].set(a_r2[0]))
    R1 = jnp.kron(eye8, jnp.ones((1, HD), jnp.float32))
    R2 = jnp.ones((1, F), jnp.float32)

    h1, elr1 = _dense1(features, W1, A1)
    el1 = elr1[:, :NH]
    er1 = elr1[:, NH:]
    elh = jnp.concatenate([el1[:, :HH].reshape(-1), el1[:, HH:].reshape(-1)])
    erh = jnp.concatenate([er1[:, :HH].reshape(-1), er1[:, HH:].reshape(-1)])
    w1, s4p = _wpass1(elh, erh, src, dst)
    s4m = s4p.reshape(NW, HH * N)
    s4a, s4b = _red1(s4m[:NS], s4m[NS:])
    s1 = jnp.concatenate([s4a.reshape(N, HH), s4b.reshape(N, HH)], axis=1)
    msg1t = _main1(h1.T.reshape(-1), w1, src, dst)
    msg1 = msg1t.reshape(F, N).T
    h2, elr2 = _dense2(msg1, s1, R1, b1.reshape(1, F), W2, A2)
    msg2t, s2p = _main2(h2.T.reshape(-1), elr2[:, 0], elr2[:, 1], src, dst)
    msg2 = msg2t.reshape(F, N).T
    s2 = s2p[:N]
    hf = _dense3(msg2, s2.reshape(N, 1), R2, b2.reshape(1, F))
    return _score(hf, src, dst)


# speedup vs baseline: 14.4809x; 1.0411x over previous
import jax
import jax.numpy as jnp
from jax import lax
from jax.experimental import pallas as pl
from jax.experimental.pallas import tpu as pltpu
from jax.experimental.pallas import tpu_sc as plsc

N = 10000
E = 320000
F = 128
NH = 8
HD = 16
L = 16
NC, NS = 2, 16
NW = NC * NS
EPW = E // NW
SEPW = E // NS
CH = 80
CH2 = 40
NIT = EPW // CH
NIT2 = EPW // CH2
NITW = SEPW // CH
NP = 10240
RPW = NP // NS
_OFFS = tuple(range(0, RPW, CH))
_OFFS2 = tuple(range(0, RPW, CH2))
HH = NH // NC
_EPS = 1e-9

_MESH = plsc.VectorSubcoreMesh(core_axis_name="c", subcore_axis_name="s",
                               num_cores=NC, num_subcores=NS)
_SC_PARAMS = pltpu.CompilerParams(needs_layout_passes=False)


def _expleaky(e):
    return jnp.exp(jnp.where(e >= 0.0, e, 0.2 * e))


def _w1_body(elh_hbm, erh_hbm, src_hbm, dst_hbm, w_out, s_out,
             el_v, er_v, s4_v, src_v, dst_v, w_v, sem):
    cid = lax.axis_index("c")
    sid = lax.axis_index("s")
    iota = lax.iota(jnp.int32, L)
    zf = (iota * 0).astype(jnp.float32)
    tb = cid * (HH * N)
    pltpu.sync_copy(elh_hbm.at[pl.ds(tb, HH * N)], el_v)
    pltpu.sync_copy(erh_hbm.at[pl.ds(tb, HH * N)], er_v)

    def _zs(i, _):
        s4_v[pl.ds(i * L, L)] = zf
        return 0
    lax.fori_loop(0, HH * N // L, _zs, 0)
    base = sid * SEPW

    def _iter(j, _):
        eb = base + j * CH
        c1 = pltpu.async_copy(src_hbm.at[pl.ds(eb, CH)], src_v, sem)
        c2 = pltpu.async_copy(dst_hbm.at[pl.ds(eb, CH)], dst_v, sem)
        c1.wait()
        c2.wait()
        for q in range(CH * HH // L):
            p0 = q * L
            le = p0 // HH + iota // HH
            lh = iota % HH
            sg = plsc.load_gather(src_v, [le])
            dg = plsc.load_gather(dst_v, [le])
            elv = plsc.load_gather(el_v, [sg * HH + lh])
            erv = plsc.load_gather(er_v, [dg * HH + lh])
            w = _expleaky(elv + erv)
            w_v[pl.ds(p0, L)] = w
            didx = dg * HH + lh
            for e4 in range(HH):
                plsc.addupdate_scatter(s4_v, [didx], w,
                                       mask=iota // HH == e4)
        pltpu.sync_copy(w_v, w_out.at[pl.ds(cid * (HH * E) + eb * HH,
                                            HH * CH)])
        return 0
    lax.fori_loop(0, NITW, _iter, 0)
    wid = cid * NS + sid
    pltpu.sync_copy(s4_v, s_out.at[pl.ds(wid * (HH * N), HH * N)])


_wpass1 = pl.kernel(
    _w1_body,
    out_type=(jax.ShapeDtypeStruct((NC * HH * E,), jnp.float32),
              jax.ShapeDtypeStruct((NW * HH * N,), jnp.float32)),
    mesh=_MESH,
    compiler_params=_SC_PARAMS,
    scratch_types=[
        pltpu.VMEM((HH * N,), jnp.float32),
        pltpu.VMEM((HH * N,), jnp.float32),
        pltpu.VMEM((HH * N,), jnp.float32),
        pltpu.VMEM((CH,), jnp.int32),
        pltpu.VMEM((CH,), jnp.int32),
        pltpu.VMEM((HH * CH,), jnp.float32),
        pltpu.SemaphoreType.DMA,
    ],
    name="gat_w1",
)


CHE = 3200
NITE = E // CHE


def _main1_body(h1t_hbm, w_hbm, src_hbm, dst_hbm, msgt_out,
                ht_v, acc_v, src_v, dst_v, w_v, sem):
    cid = lax.axis_index("c")
    sid = lax.axis_index("s")
    wid = sid * NC + cid
    head = wid // 4
    hw = head % HH
    hf_sel = head // HH
    iota = lax.iota(jnp.int32, L)
    zf = (iota * 0).astype(jnp.float32)
    pltpu.sync_copy(h1t_hbm.at[pl.ds(wid * (4 * N), 4 * N)], ht_v)

    def _za(i, _):
        acc_v[pl.ds(i * L, L)] = zf
        return 0
    lax.fori_loop(0, 4 * N // L, _za, 0)

    def _iter(j, _):
        eb = j * CHE
        c1 = pltpu.async_copy(src_hbm.at[pl.ds(eb, CHE)], src_v, sem)
        c2 = pltpu.async_copy(dst_hbm.at[pl.ds(eb, CHE)], dst_v, sem)
        c3 = pltpu.async_copy(
            w_hbm.at[pl.ds(hf_sel * (HH * E) + HH * eb, HH * CHE)], w_v, sem)
        c1.wait()
        c2.wait()
        c3.wait()
        for g in range(CHE // L):
            j0 = g * L
            sg = plsc.load_gather(src_v, [j0 + iota])
            dg = plsc.load_gather(dst_v, [j0 + iota])
            wv = plsc.load_gather(w_v, [HH * (j0 + iota) + hw])
            for cl in range(4):
                hv = plsc.load_gather(ht_v, [cl * N + sg])
                plsc.addupdate_scatter(acc_v, [cl * N + dg], hv * wv)
        return 0
    lax.fori_loop(0, NITE, _iter, 0)
    pltpu.sync_copy(acc_v, msgt_out.at[pl.ds(wid * (4 * N), 4 * N)])


_main1 = pl.kernel(
    _main1_body,
    out_type=jax.ShapeDtypeStruct((F * N,), jnp.float32),
    mesh=_MESH,
    compiler_params=_SC_PARAMS,
    scratch_types=[
        pltpu.VMEM((4 * N,), jnp.float32),
        pltpu.VMEM((4 * N,), jnp.float32),
        pltpu.VMEM((CHE,), jnp.int32),
        pltpu.VMEM((CHE,), jnp.int32),
        pltpu.VMEM((HH * CHE,), jnp.float32),
        pltpu.SemaphoreType.DMA,
    ],
    name="gat_main1",
)


def _main2_body(h2t_hbm, el_hbm, er_hbm, src_hbm, dst_hbm, msgt_out, s_out,
                ht_v, acc_v, el_v, er_v, s2_v, src_v, dst_v, sem):
    cid = lax.axis_index("c")
    sid = lax.axis_index("s")
    wid = sid * NC + cid
    iota = lax.iota(jnp.int32, L)
    zf = (iota * 0).astype(jnp.float32)
    pltpu.sync_copy(h2t_hbm.at[pl.ds(wid * (4 * N), 4 * N)], ht_v)
    pltpu.sync_copy(el_hbm, el_v)
    pltpu.sync_copy(er_hbm, er_v)

    def _za(i, _):
        acc_v[pl.ds(i * L, L)] = zf
        return 0
    lax.fori_loop(0, 4 * N // L, _za, 0)

    def _zs(i, _):
        s2_v[pl.ds(i * L, L)] = zf
        return 0
    lax.fori_loop(0, N // L, _zs, 0)

    def _iter(j, _):
        eb = j * CHE
        c1 = pltpu.async_copy(src_hbm.at[pl.ds(eb, CHE)], src_v, sem)
        c2 = pltpu.async_copy(dst_hbm.at[pl.ds(eb, CHE)], dst_v, sem)
        c1.wait()
        c2.wait()
        for g in range(CHE // L):
            j0 = g * L
            sg = plsc.load_gather(src_v, [j0 + iota])
            dg = plsc.load_gather(dst_v, [j0 + iota])
            w2 = _expleaky(plsc.load_gather(el_v, [sg]) +
                           plsc.load_gather(er_v, [dg]))
            plsc.addupdate_scatter(s2_v, [dg], w2)
            for cl in range(4):
                hv = plsc.load_gather(ht_v, [cl * N + sg])
                plsc.addupdate_scatter(acc_v, [cl * N + dg], hv * w2)
        return 0
    lax.fori_loop(0, NITE, _iter, 0)
    pltpu.sync_copy(acc_v, msgt_out.at[pl.ds(wid * (4 * N), 4 * N)])
    pltpu.sync_copy(s2_v, s_out.at[pl.ds(wid * N, N)])


_main2 = pl.kernel(
    _main2_body,
    out_type=(jax.ShapeDtypeStruct((F * N,), jnp.float32),
              jax.ShapeDtypeStruct((NW * N,), jnp.float32)),
    mesh=_MESH,
    compiler_params=_SC_PARAMS,
    scratch_types=[
        pltpu.VMEM((4 * N,), jnp.float32),
        pltpu.VMEM((4 * N,), jnp.float32),
        pltpu.VMEM((N,), jnp.float32),
        pltpu.VMEM((N,), jnp.float32),
        pltpu.VMEM((N,), jnp.float32),
        pltpu.VMEM((CHE,), jnp.int32),
        pltpu.VMEM((CHE,), jnp.int32),
        pltpu.SemaphoreType.DMA,
    ],
    name="gat_main2",
)


def _score_body(hf_hbm, src_hbm, dst_hbm, out_hbm,
                src_v, dst_v, a_v, b_v, dot_v, out_v, sem):
    cid = lax.axis_index("c")
    sid = lax.axis_index("s")
    base = (sid * NC + cid) * EPW
    iota = lax.iota(jnp.int32, L)

    def _iter(j, _):
        eb = base + j * CH
        i1 = pltpu.async_copy(src_hbm.at[pl.ds(eb, CH)], src_v, sem)
        i2 = pltpu.async_copy(dst_hbm.at[pl.ds(eb, CH)], dst_v, sem)
        i1.wait()
        i2.wait()
        c1 = pltpu.async_copy(hf_hbm.at[src_v], a_v, sem)
        c2 = pltpu.async_copy(hf_hbm.at[dst_v], b_v, sem)
        c1.wait()
        c2.wait()
        for g in range(CH // L):
            j0 = g * L
            for e in range(L):
                acc = a_v[j0 + e, pl.ds(0, L)] * b_v[j0 + e, pl.ds(0, L)]
                for h in range(1, F // L):
                    acc = acc + (a_v[j0 + e, pl.ds(h * L, L)] *
                                 b_v[j0 + e, pl.ds(h * L, L)])
                dot_v[e, pl.ds(0, L)] = acc
            tot = plsc.load_gather(dot_v, [iota, iota * 0])
            for k in range(1, L):
                tot = tot + plsc.load_gather(dot_v, [iota, iota * 0 + k])
            out_v[pl.ds(j0, L)] = 1.0 / (1.0 + jnp.exp(-tot))
        pltpu.sync_copy(out_v, out_hbm.at[pl.ds(eb, CH)])
        return 0
    lax.fori_loop(0, NIT, _iter, 0)


_score = pl.kernel(
    _score_body,
    out_type=jax.ShapeDtypeStruct((E,), jnp.float32),
    mesh=_MESH,
    compiler_params=_SC_PARAMS,
    scratch_types=[
        pltpu.VMEM((CH,), jnp.int32),
        pltpu.VMEM((CH,), jnp.int32),
        pltpu.VMEM((CH, F), jnp.float32),
        pltpu.VMEM((CH, F), jnp.float32),
        pltpu.VMEM((L, L), jnp.float32),
        pltpu.VMEM((CH,), jnp.float32),
        pltpu.SemaphoreType.DMA,
    ],
    name="gat_score",
)


def _dense1_body(x_ref, w_ref, a_ref, h_ref, elr_ref):
    h = jnp.dot(x_ref[...], w_ref[...], preferred_element_type=jnp.float32)
    h_ref[...] = h
    elr_ref[...] = jnp.dot(h, a_ref[...], preferred_element_type=jnp.float32)


_dense1 = pl.pallas_call(
    _dense1_body,
    out_shape=(jax.ShapeDtypeStruct((N, F), jnp.float32),
               jax.ShapeDtypeStruct((N, L), jnp.float32)),
)


def _red1_body(a_ref, b_ref, oa_ref, ob_ref):
    oa_ref[...] = jnp.sum(a_ref[...], axis=0)
    ob_ref[...] = jnp.sum(b_ref[...], axis=0)


_red1 = pl.pallas_call(
    _red1_body,
    out_shape=(jax.ShapeDtypeStruct((HH * N,), jnp.float32),
               jax.ShapeDtypeStruct((HH * N,), jnp.float32)),
)


def _dense2_body(msg_ref, s_ref, r_ref, b_ref, w_ref, a_ref, h_ref, elr_ref):
    msg = msg_ref[...]
    srep = jnp.dot(s_ref[...], r_ref[...], preferred_element_type=jnp.float32)
    x2 = jnp.maximum(msg / (srep + _EPS) + b_ref[...], 0.0)
    h2 = jnp.dot(x2, w_ref[...], preferred_element_type=jnp.float32)
    h_ref[...] = h2
    elr_ref[...] = jnp.dot(h2, a_ref[...], preferred_element_type=jnp.float32)


_dense2 = pl.pallas_call(
    _dense2_body,
    out_shape=(jax.ShapeDtypeStruct((N, F), jnp.float32),
               jax.ShapeDtypeStruct((N, L), jnp.float32)),
)


def _dense3_body(msg_ref, s_ref, r_ref, b_ref, hf_ref):
    msg = msg_ref[...]
    srep = jnp.dot(s_ref[...], r_ref[...], preferred_element_type=jnp.float32)
    hf_ref[...] = jnp.maximum(msg / (srep + _EPS) + b_ref[...], 0.0)


_dense3 = pl.pallas_call(
    _dense3_body,
    out_shape=jax.ShapeDtypeStruct((N, F), jnp.float32),
)


def kernel(features, edge_index, edge_type, W1, a_l1, a_r1, b1,
           W2, a_l2, a_r2, b2):
    del edge_type
    src = edge_index[0]
    dst = edge_index[1]
    eye8 = jnp.eye(NH, dtype=jnp.float32)
    Al = (a_l1[:, :, None] * eye8[:, None, :]).reshape(F, NH)
    Ar = (a_r1[:, :, None] * eye8[:, None, :]).reshape(F, NH)
    A1 = jnp.concatenate([Al, Ar], axis=1)
    A2 = (jnp.zeros((F, L), jnp.float32)
          .at[:, 0].set(a_l2[0]).at[:, 1].set(a_r2[0]))
    R1 = jnp.kron(eye8, jnp.ones((1, HD), jnp.float32))
    R2 = jnp.ones((1, F), jnp.float32)

    h1, elr1 = _dense1(features, W1, A1)
    el1 = elr1[:, :NH]
    er1 = elr1[:, NH:]
    elh = jnp.concatenate([el1[:, :HH].reshape(-1), el1[:, HH:].reshape(-1)])
    erh = jnp.concatenate([er1[:, :HH].reshape(-1), er1[:, HH:].reshape(-1)])
    w1, s4p = _wpass1(elh, erh, src, dst)
    s4m = s4p.reshape(NW, HH * N)
    s4a, s4b = _red1(s4m[:NS], s4m[NS:])
    s1 = jnp.concatenate([s4a.reshape(N, HH), s4b.reshape(N, HH)], axis=1)
    msg1t = _main1(h1.T.reshape(-1), w1, src, dst)
    msg1 = msg1t.reshape(F, N).T
    h2, elr2 = _dense2(msg1, s1, R1, b1.reshape(1, F), W2, A2)
    msg2t, s2p = _main2(h2.T.reshape(-1), elr2[:, 0], elr2[:, 1], src, dst)
    msg2 = msg2t.reshape(F, N).T
    s2 = s2p[:N]
    hf = _dense3(msg2, s2.reshape(N, 1), R2, b2.reshape(1, F))
    return _score(hf, src, dst)
